# Initial kernel scaffold; baseline (speedup 1.0000x reference)
import jax
import jax.numpy as jnp
from jax.experimental import pallas as pl


def kernel(events, W1, b1, W2, b2):
    raise NotImplementedError("write your pallas kernel here")


# speedup vs baseline: 20.2743x; 20.2743x over previous
import functools

import jax
import jax.numpy as jnp
from jax import lax
from jax.experimental import pallas as pl
from jax.experimental.pallas import tpu as pltpu
from jax.experimental.pallas import tpu_sc as plsc

K = 16
RB = 512
RB3 = 512
SC_CORES = 2
SC_SUBCORES = 16
CH = 128


def _topk_proj_body(cxr_ref, cyr_ref, cxc_ref, cyc_ref, e_ref, w1_ref, b1_ref,
                    idx_ref, a_ref, bv_ref):
    n = cxr_ref.shape[2]
    f = e_ref.shape[2]
    h = w1_ref.shape[1]
    dx = cxr_ref[0] - cxc_ref[0]
    dy = cyr_ref[0] - cyc_ref[0]
    d = jnp.sqrt(dx * dx + dy * dy)
    iot = lax.broadcasted_iota(jnp.int32, d.shape, 1)
    inf = jnp.float32(jnp.inf)
    self_ids = (lax.broadcasted_iota(jnp.int32, (d.shape[0], 1), 0)
                + pl.program_id(1) * RB)
    d = jnp.where(iot == self_ids, inf, d)
    for t in range(K):
        idx = jnp.argmin(d, axis=1, keepdims=True).astype(jnp.int32)
        idx_ref[0, :, pl.ds(t, 1)] = idx
        d = jnp.where(iot == idx, inf, d)
    e = e_ref[0]
    w1 = w1_ref[...]
    w_top = w1[0:f, :]
    w_bot = w1[f:2 * f, :]
    a = lax.dot_general(e, w_top - w_bot, (((1,), (0,)), ((), ())),
                        preferred_element_type=jnp.float32)
    a_ref[0] = a + b1_ref[...]
    bv_ref[0] = lax.dot_general(w_bot, e, (((0,), (1,)), ((), ())),
                                preferred_element_type=jnp.float32)


def _build_graph(events, W1, b1):
    B, N, F = events.shape
    H = W1.shape[1]
    cx = events[:, :, 0]
    cy = events[:, :, 1]
    grid = (B, N // RB)
    out_shape = [
        jax.ShapeDtypeStruct((B, N, K), jnp.int32),
        jax.ShapeDtypeStruct((B, N, H), jnp.float32),
        jax.ShapeDtypeStruct((B, H, N), jnp.float32),
    ]
    in_specs = [
        pl.BlockSpec((1, 1, N), lambda b, r: (b, 0, 0)),
        pl.BlockSpec((1, 1, N), lambda b, r: (b, 0, 0)),
        pl.BlockSpec((1, RB, 1), lambda b, r: (b, r, 0)),
        pl.BlockSpec((1, RB, 1), lambda b, r: (b, r, 0)),
        pl.BlockSpec((1, RB, F), lambda b, r: (b, r, 0)),
        pl.BlockSpec((2 * F, H), lambda b, r: (0, 0)),
        pl.BlockSpec((1, H), lambda b, r: (0, 0)),
    ]
    out_specs = [
        pl.BlockSpec((1, RB, K), lambda b, r: (b, r, 0)),
        pl.BlockSpec((1, RB, H), lambda b, r: (b, r, 0)),
        pl.BlockSpec((1, H, RB), lambda b, r: (b, 0, r)),
    ]
    return pl.pallas_call(
        _topk_proj_body, grid=grid, in_specs=in_specs, out_specs=out_specs,
        out_shape=out_shape,
    )(cx[:, None, :], cy[:, None, :], cx[:, :, None], cy[:, :, None],
      events, W1, b1[None, :])


def _sc_edge_mean(idx_flat, a_flat, bv_flat, N, H):
    B = idx_flat.shape[0]
    NW = SC_CORES * SC_SUBCORES
    wpb = NW // B
    nodes_per_w = N // wpb
    ch = min(CH, nodes_per_w)
    n_chunks = nodes_per_w // ch
    mesh = plsc.VectorSubcoreMesh(
        core_axis_name="c", subcore_axis_name="s",
        num_cores=SC_CORES, num_subcores=SC_SUBCORES)

    @functools.partial(
        pl.kernel, mesh=mesh,
        compiler_params=pltpu.CompilerParams(needs_layout_passes=False),
        out_type=jax.ShapeDtypeStruct((B, N * H), jnp.float32),
        scratch_types=[
            pltpu.VMEM((N * H,), jnp.float32),
            pltpu.VMEM((ch * K,), jnp.int32),
            pltpu.VMEM((ch * H,), jnp.float32),
            pltpu.VMEM((ch * H,), jnp.float32),
            pltpu.SemaphoreType.DMA,
            pltpu.SemaphoreType.DMA,
            pltpu.SemaphoreType.DMA,
        ],
    )
    def k(idx_hbm, a_hbm, bv_hbm, out_hbm, bv_v, idx_v, a_v, s_v,
          sem_bv, sem_idx, sem_a):
        wid = lax.axis_index("s") * SC_CORES + lax.axis_index("c")
        b = wid // wpb
        part = wid % wpb
        cp_bv = pltpu.async_copy(bv_hbm.at[b], bv_v, sem_bv)

        def chunk_body(c, _):
            n0 = part * nodes_per_w + c * ch
            cp_idx = pltpu.async_copy(
                idx_hbm.at[b, pl.ds(n0 * K, ch * K)], idx_v, sem_idx)
            cp_a = pltpu.async_copy(
                a_hbm.at[b, pl.ds(n0 * H, ch * H)], a_v, sem_a)
            @pl.when(c == 0)
            def _():
                cp_bv.wait()

            cp_idx.wait()
            cp_a.wait()

            def group_body(g, _):
                rows = lax.broadcasted_iota(jnp.int32, (16,), 0) + g * 16
                rows_k = rows * K
                rows_h = rows * H
                bases = [plsc.load_gather(idx_v, [rows_k + j])
                         for j in range(K)]
                for f in range(H):
                    nf = rows_h + f
                    a_f = plsc.load_gather(a_v, [nf])
                    bv_f = bv_v.at[pl.ds(f * N, N)]
                    t = [jnp.maximum(a_f + plsc.load_gather(bv_f, [bases[j]]),
                                     0.0)
                         for j in range(K)]
                    while len(t) > 1:
                        t = [t[i] + t[i + 1] for i in range(0, len(t) - 1, 2)] \
                            + ([t[-1]] if len(t) % 2 else [])
                    plsc.store_scatter(s_v, [nf], t[0])
                return 0

            lax.fori_loop(0, ch // 16, group_body, 0)
            pltpu.sync_copy(s_v, out_hbm.at[b, pl.ds(n0 * H, ch * H)])
            return 0

        lax.fori_loop(0, n_chunks, chunk_body, 0)

    return k(idx_flat, a_flat, bv_flat)


def _final_body(s_ref, w2_ref, b2_ref, o_ref):
    s = s_ref[0]
    o = lax.dot_general(s, w2_ref[...], (((1,), (0,)), ((), ())),
                        preferred_element_type=jnp.float32)
    o_ref[0] = o * jnp.float32(1.0 / K) + b2_ref[...]


def _final_proj(s, W2, b2):
    B, N, H = s.shape
    OUT = W2.shape[1]
    grid = (B, N // RB3)
    return pl.pallas_call(
        _final_body, grid=grid,
        in_specs=[
            pl.BlockSpec((1, RB3, H), lambda b, r: (b, r, 0)),
            pl.BlockSpec((H, OUT), lambda b, r: (0, 0)),
            pl.BlockSpec((1, OUT), lambda b, r: (0, 0)),
        ],
        out_specs=pl.BlockSpec((1, RB3, OUT), lambda b, r: (b, r, 0)),
        out_shape=jax.ShapeDtypeStruct((B, N, OUT), jnp.float32),
    )(s, W2, b2[None, :])


NSLICE = 4


def kernel(events, W1, b1, W2, b2):
    B, N, F = events.shape
    H = W1.shape[1]
    bs = B // NSLICE
    outs = []
    for si in range(NSLICE):
        ev = events[si * bs:(si + 1) * bs]
        idx, a, bv = _build_graph(ev, W1, b1)
        s = _sc_edge_mean(idx.reshape(bs, N * K), a.reshape(bs, N * H),
                          bv.reshape(bs, H * N), N, H)
        outs.append(_final_proj(s.reshape(bs, N, H), W2, b2))
    return jnp.concatenate(outs, axis=0)
